# Initial kernel scaffold; baseline (speedup 1.0000x reference)
import jax
import jax.numpy as jnp
from jax.experimental import pallas as pl


def kernel(x0, t, noise, sqrt_alphas_cumprod, sqrt_one_minus_alphas_cumprod):
    raise NotImplementedError("write your pallas kernel here")


# speedup vs baseline: 3.7200x; 3.7200x over previous
import functools

import jax
import jax.numpy as jnp
from jax import lax
from jax.experimental import pallas as pl
from jax.experimental.pallas import tpu as pltpu
from jax.experimental.pallas import tpu_sc as plsc

B, D, T = 16384, 128, 1000
NW = 32
ROWS_PER_W = B // NW
CH = 128
NCHUNK = ROWS_PER_W // CH
LANES = 16


def _body(x0_hbm, t_hbm, noise_hbm, ta_hbm, tb_hbm, out_hbm,
          ta_v, tb_v, t_v, x0_v, nz_v, out_v):
    wid = lax.axis_index("s") * 2 + lax.axis_index("c")

    pltpu.sync_copy(ta_hbm, ta_v)
    pltpu.sync_copy(tb_hbm, tb_v)

    for c in range(NCHUNK):
        base = wid * ROWS_PER_W + c * CH
        pltpu.sync_copy(t_hbm.at[pl.ds(base, CH)], t_v)
        pltpu.sync_copy(x0_hbm.at[pl.ds(base, CH)], x0_v)
        pltpu.sync_copy(noise_hbm.at[pl.ds(base, CH)], nz_v)

        def row_group(g, _):
            idx = t_v[pl.ds(g * LANES, LANES)]
            ca = plsc.load_gather(ta_v, [idx])
            cb = plsc.load_gather(tb_v, [idx])
            for i in range(LANES):
                r = g * LANES + i
                a = ca[i]
                b = cb[i]
                for j in range(D // LANES):
                    sl = pl.ds(j * LANES, LANES)
                    out_v[r, sl] = a * x0_v[r, sl] + b * nz_v[r, sl]
            return 0

        lax.fori_loop(0, CH // LANES, row_group, 0)
        pltpu.sync_copy(out_v, out_hbm.at[pl.ds(base, CH)])


def kernel(x0, t, noise, sqrt_alphas_cumprod, sqrt_one_minus_alphas_cumprod):
    mesh = plsc.VectorSubcoreMesh(core_axis_name="c", subcore_axis_name="s")
    f = functools.partial(
        pl.kernel,
        mesh=mesh,
        out_type=jax.ShapeDtypeStruct((B, D), jnp.float32),
        compiler_params=pltpu.CompilerParams(needs_layout_passes=False),
        scratch_types=[
            pltpu.VMEM((T,), jnp.float32),
            pltpu.VMEM((T,), jnp.float32),
            pltpu.VMEM((CH,), jnp.int32),
            pltpu.VMEM((CH, D), jnp.float32),
            pltpu.VMEM((CH, D), jnp.float32),
            pltpu.VMEM((CH, D), jnp.float32),
        ],
    )(_body)
    return f(x0, t, noise, sqrt_alphas_cumprod, sqrt_one_minus_alphas_cumprod)


# speedup vs baseline: 4.8772x; 1.3111x over previous
import functools

import jax
import jax.numpy as jnp
from jax import lax
from jax.experimental import pallas as pl
from jax.experimental.pallas import tpu as pltpu
from jax.experimental.pallas import tpu_sc as plsc

B, D, T = 16384, 128, 1000
NW = 32
ROWS_PER_W = B // NW
CH = 128
NCHUNK = ROWS_PER_W // CH
LANES = 16


def _body(x0_hbm, t_hbm, noise_hbm, ta_hbm, tb_hbm, out_hbm,
          ta_v, tb_v, t_v, x0_v, nz_v, out_v,
          sem_tab, sem_in, sem_out):
    wid = lax.axis_index("s") * 2 + lax.axis_index("c")

    htab_a = pltpu.async_copy(ta_hbm, ta_v, sem_tab)
    htab_b = pltpu.async_copy(tb_hbm, tb_v, sem_tab)

    def start_in(c):
        slot = c % 2
        base = wid * ROWS_PER_W + c * CH
        return (
            pltpu.async_copy(t_hbm.at[pl.ds(base, CH)], t_v.at[slot],
                             sem_in.at[slot]),
            pltpu.async_copy(x0_hbm.at[pl.ds(base, CH)], x0_v.at[slot],
                             sem_in.at[slot]),
            pltpu.async_copy(noise_hbm.at[pl.ds(base, CH)], nz_v.at[slot],
                             sem_in.at[slot]),
        )

    def start_out(c):
        slot = c % 2
        base = wid * ROWS_PER_W + c * CH
        return pltpu.async_copy(out_v.at[slot], out_hbm.at[pl.ds(base, CH)],
                                sem_out.at[slot])

    def compute(slot):
        def row_group(g, _):
            idx = t_v[slot, pl.ds(g * LANES, LANES)]
            ca = plsc.load_gather(ta_v, [idx])
            cb = plsc.load_gather(tb_v, [idx])
            for i in range(LANES):
                r = g * LANES + i
                a = ca[i]
                b = cb[i]
                for j in range(D // LANES):
                    sl = pl.ds(j * LANES, LANES)
                    out_v[slot, r, sl] = (a * x0_v[slot, r, sl]
                                          + b * nz_v[slot, r, sl])
            return 0

        lax.fori_loop(0, CH // LANES, row_group, 0)

    in_handles = {0: start_in(0)}
    out_handles = {}
    htab_a.wait()
    htab_b.wait()
    for c in range(NCHUNK):
        if c + 1 < NCHUNK:
            in_handles[c + 1] = start_in(c + 1)
        for h in in_handles.pop(c):
            h.wait()
        if c >= 2:
            out_handles.pop(c - 2).wait()
        compute(c % 2)
        out_handles[c] = start_out(c)
    for h in out_handles.values():
        h.wait()


def kernel(x0, t, noise, sqrt_alphas_cumprod, sqrt_one_minus_alphas_cumprod):
    mesh = plsc.VectorSubcoreMesh(core_axis_name="c", subcore_axis_name="s")
    f = functools.partial(
        pl.kernel,
        mesh=mesh,
        out_type=jax.ShapeDtypeStruct((B, D), jnp.float32),
        compiler_params=pltpu.CompilerParams(needs_layout_passes=False),
        scratch_types=[
            pltpu.VMEM((T,), jnp.float32),
            pltpu.VMEM((T,), jnp.float32),
            pltpu.VMEM((2, CH), jnp.int32),
            pltpu.VMEM((2, CH, D), jnp.float32),
            pltpu.VMEM((2, CH, D), jnp.float32),
            pltpu.VMEM((2, CH, D), jnp.float32),
            pltpu.SemaphoreType.DMA,
            pltpu.SemaphoreType.DMA((2,)),
            pltpu.SemaphoreType.DMA((2,)),
        ],
    )(_body)
    return f(x0, t, noise, sqrt_alphas_cumprod, sqrt_one_minus_alphas_cumprod)


# speedup vs baseline: 6.1737x; 1.2658x over previous
import functools

import jax
import jax.numpy as jnp
from jax import lax
from jax.experimental import pallas as pl
from jax.experimental.pallas import tpu as pltpu
from jax.experimental.pallas import tpu_sc as plsc

B, D, T = 16384, 128, 1000
NW = 32
ROWS_PER_W = B // NW
CH = 128
NCHUNK = ROWS_PER_W // CH
LANES = 16
GROUPS = CH // LANES


def _body(x0_hbm, t_hbm, noise_hbm, ta_hbm, tb_hbm, out_hbm,
          ta_v, tb_v, t_v, ca_v, cb_v, x0_v, nz_v, out_v,
          sem_tab, sem_in, sem_out):
    wid = lax.axis_index("s") * 2 + lax.axis_index("c")
    slab = wid * ROWS_PER_W

    htab_a = pltpu.async_copy(ta_hbm, ta_v, sem_tab)
    htab_b = pltpu.async_copy(tb_hbm, tb_v, sem_tab)
    ht = pltpu.async_copy(t_hbm.at[pl.ds(slab, ROWS_PER_W)], t_v, sem_tab)

    def start_in(c):
        slot = c % 2
        base = slab + c * CH
        return (
            pltpu.async_copy(x0_hbm.at[pl.ds(base, CH)], x0_v.at[slot],
                             sem_in.at[slot]),
            pltpu.async_copy(noise_hbm.at[pl.ds(base, CH)], nz_v.at[slot],
                             sem_in.at[slot]),
        )

    def start_out(c):
        slot = c % 2
        base = slab + c * CH
        return pltpu.async_copy(out_v.at[slot], out_hbm.at[pl.ds(base, CH)],
                                sem_out.at[slot])

    in_handles = {0: start_in(0)}
    htab_a.wait()
    htab_b.wait()
    ht.wait()

    def gather_grp(g, _):
        sl = pl.ds(g * LANES, LANES)
        idx = t_v[sl]
        ca_v[sl] = plsc.load_gather(ta_v, [idx])
        cb_v[sl] = plsc.load_gather(tb_v, [idx])
        return 0

    lax.fori_loop(0, ROWS_PER_W // LANES, gather_grp, 0)

    in_handles[1] = start_in(1)
    out_handles = {}

    def compute(c):
        slot = c % 2
        x0s, nzs, outs = x0_v.at[slot], nz_v.at[slot], out_v.at[slot]

        def row_group(g, _):
            for i in range(LANES):
                r = g * LANES + i
                rsplat = jnp.broadcast_to(c * CH + r, (LANES,))
                av = plsc.load_gather(ca_v, [rsplat])
                bv = plsc.load_gather(cb_v, [rsplat])
                xr, nr, outr = x0s.at[r], nzs.at[r], outs.at[r]
                for j in range(D // LANES):
                    sl = pl.ds(j * LANES, LANES)
                    outr[sl] = av * xr[sl] + bv * nr[sl]
            return 0

        lax.fori_loop(0, GROUPS, row_group, 0)

    for c in range(NCHUNK):
        for h in in_handles.pop(c):
            h.wait()
        if c + 2 < NCHUNK:
            in_handles[c + 2] = start_in(c + 2)
        if c >= 2:
            out_handles.pop(c - 2).wait()
        compute(c)
        out_handles[c] = start_out(c)
    for h in out_handles.values():
        h.wait()


def kernel(x0, t, noise, sqrt_alphas_cumprod, sqrt_one_minus_alphas_cumprod):
    mesh = plsc.VectorSubcoreMesh(core_axis_name="c", subcore_axis_name="s")
    f = functools.partial(
        pl.kernel,
        mesh=mesh,
        out_type=jax.ShapeDtypeStruct((B, D), jnp.float32),
        compiler_params=pltpu.CompilerParams(needs_layout_passes=False),
        scratch_types=[
            pltpu.VMEM((T,), jnp.float32),
            pltpu.VMEM((T,), jnp.float32),
            pltpu.VMEM((ROWS_PER_W,), jnp.int32),
            pltpu.VMEM((ROWS_PER_W,), jnp.float32),
            pltpu.VMEM((ROWS_PER_W,), jnp.float32),
            pltpu.VMEM((2, CH, D), jnp.float32),
            pltpu.VMEM((2, CH, D), jnp.float32),
            pltpu.VMEM((2, CH, D), jnp.float32),
            pltpu.SemaphoreType.DMA,
            pltpu.SemaphoreType.DMA((2,)),
            pltpu.SemaphoreType.DMA((2,)),
        ],
    )(_body)
    return f(x0, t, noise, sqrt_alphas_cumprod, sqrt_one_minus_alphas_cumprod)
